# baseline (speedup 1.0000x reference)
import functools

import jax
import jax.numpy as jnp
from jax import lax
from jax.experimental import pallas as pl
from jax.experimental.pallas import tpu as pltpu
from jax.experimental.pallas import tpu_sc as plsc

_NUM_FIELDS = 26
_VOCAB = 100000
_EMBED_DIM = 16
_BATCH = 16384

_ROWS = _BATCH * _NUM_FIELDS
_NC = 2
_NS = 16
_NW = _NC * _NS
_PER_W = _ROWS // _NW
_CH = 6656
_NCH = _PER_W // _CH
_LANES = 16


def _body(xf_hbm, tab_hbm, out_hbm, idx_v, rows_v, sem):
    wid = lax.axis_index("s") * _NC + lax.axis_index("c")
    base_w = wid * _PER_W

    pltpu.sync_copy(xf_hbm.at[pl.ds(base_w, _PER_W)], idx_v)
    lane = lax.iota(jnp.int32, _LANES)

    def fix(i, _):
        sl = pl.ds(i * _LANES, _LANES)
        r = (base_w + i * _LANES) + lane
        f = lax.rem(r, _NUM_FIELDS)
        idx_v[sl] = idx_v[sl] + f * _VOCAB
        return 0

    lax.fori_loop(0, _PER_W // _LANES, fix, 0)

    def chunk(c, _):
        base = base_w + c * _CH
        pltpu.async_copy(
            tab_hbm.at[idx_v.at[pl.ds(c * _CH, _CH)]], rows_v, sem
        ).wait()
        pltpu.sync_copy(rows_v, out_hbm.at[pl.ds(base, _CH)])
        return 0

    lax.fori_loop(0, _NCH, chunk, 0)


@functools.partial(jax.jit, static_argnames=())
def _gather(xf, tab):
    mesh = plsc.VectorSubcoreMesh(core_axis_name="c", subcore_axis_name="s")
    return pl.kernel(
        _body,
        out_type=jax.ShapeDtypeStruct((_ROWS, _EMBED_DIM), jnp.float32),
        mesh=mesh,
        compiler_params=pltpu.CompilerParams(use_tc_tiling_on_sc=False),
        scratch_types=[
            pltpu.VMEM((_PER_W,), jnp.int32),
            pltpu.VMEM((_CH, _EMBED_DIM), jnp.float32),
            pltpu.SemaphoreType.DMA,
        ],
    )(xf, tab)


def kernel(x, edge_index, tables):
    del edge_index
    xf = x.reshape(-1).astype(jnp.int32)
    tab = tables.reshape(_NUM_FIELDS * _VOCAB, _EMBED_DIM)
    out = _gather(xf, tab)
    return out.reshape(_BATCH, _NUM_FIELDS * _EMBED_DIM)
